# baseline (speedup 1.0000x reference)
import jax
import jax.numpy as jnp
from jax import lax
from jax.experimental import pallas as pl

B, S, K, D = 4096, 20, 64, 32
BB = 128


def _body(s_ref, m_ref, k_ref, ut_ref, vt_ref, wt_ref, o_ref):
    f32 = jnp.float32
    kt = k_ref[...].reshape(D, K * BB)
    ut = ut_ref[...]
    vt = vt_ref[...]
    wt = wt_ref[...]

    kv = jnp.dot(vt, kt, preferred_element_type=f32)
    kt3 = kt.reshape(D, K, BB)
    kv3 = kv.reshape(D, K, BB)
    half = jnp.full((1, D), 0.5, f32)

    def step(t, h3):
        s_t = s_ref[pl.ds(t, 1)].reshape(D, BB)
        m_t = m_ref[pl.ds(t, 1)].reshape(1, 1, BB)
        x = s_t[:, None, :] * (h3 + kt3)
        g_half = jnp.dot(half, x.reshape(D, K * BB),
                         preferred_element_type=f32)
        g4 = 0.25 * (jnp.tanh(g_half) + 1.0)
        g4 = g4.reshape(1, K, BB)
        hu = jnp.dot(ut, h3.reshape(D, K * BB),
                     preferred_element_type=f32)
        sw = jnp.dot(wt, s_t, preferred_element_type=f32)
        z = hu.reshape(D, K, BB) + kv3 + sw[:, None, :]
        upd = h3 + g4 * (jnp.tanh(z) + 1.0)
        sq = jnp.dot(jnp.full((1, D), 1.0, f32),
                     (upd * upd).reshape(D, K * BB),
                     preferred_element_type=f32).reshape(1, K, BB)
        upd = upd * lax.rsqrt(jnp.maximum(sq, 1e-12))
        return jnp.where(m_t > 0.5, upd, h3)

    o_ref[...] = lax.fori_loop(0, S, step, kt3)


@jax.jit
def kernel(encoded_sents, mask, keys, U, V, W):
    s_t = jnp.transpose(encoded_sents, (1, 2, 0))
    m_t = jnp.swapaxes(mask, 0, 1).astype(jnp.float32)[:, None, :]
    k_t = jnp.transpose(keys, (2, 1, 0))
    grid = (B // BB,)
    out_t = pl.pallas_call(
        _body,
        grid=grid,
        in_specs=[
            pl.BlockSpec((S, D, BB), lambda i: (0, 0, i)),
            pl.BlockSpec((S, 1, BB), lambda i: (0, 0, i)),
            pl.BlockSpec((D, K, BB), lambda i: (0, 0, i)),
            pl.BlockSpec((D, D), lambda i: (0, 0)),
            pl.BlockSpec((D, D), lambda i: (0, 0)),
            pl.BlockSpec((D, D), lambda i: (0, 0)),
        ],
        out_specs=pl.BlockSpec((D, K, BB), lambda i: (0, 0, i)),
        out_shape=jax.ShapeDtypeStruct((D, K, B), jnp.float32),
    )(s_t, m_t, k_t, 0.5 * U.T, 0.5 * V.T, 0.5 * W.T)
    return jnp.transpose(out_t, (2, 1, 0))


# speedup vs baseline: 1.0215x; 1.0215x over previous
import jax
import jax.numpy as jnp
from jax import lax
from jax.experimental import pallas as pl

B, S, K, D = 4096, 20, 64, 32
BB = 128
NBLK = B // BB
N = K * BB


def _body(s_ref, m_ref, k_ref, ut_ref, vt_ref, wt_ref, o_ref):
    f32 = jnp.float32
    kt = k_ref[0]
    ut = ut_ref[...]
    vt = vt_ref[...]
    wt = wt_ref[...]

    kv = jnp.dot(vt, kt, preferred_element_type=f32)
    half = jnp.full((1, D), 0.5, f32)
    ones = jnp.full((1, D), 1.0, f32)
    tmat = (lax.broadcasted_iota(jnp.int32, (BB, N), 1) % BB
            == lax.broadcasted_iota(jnp.int32, (BB, N), 0)).astype(f32)

    def step(t, h):
        s_t = s_ref[pl.ds(t, 1)].reshape(D, BB)
        m_t = m_ref[pl.ds(t, 1)].reshape(1, N)
        s_tiled = jnp.dot(s_t, tmat, preferred_element_type=f32)
        g_half = jnp.dot(half, s_tiled * (h + kt),
                         preferred_element_type=f32)
        g4 = 0.25 * jnp.tanh(g_half) + 0.25
        hu = jnp.dot(ut, h, preferred_element_type=f32)
        sw = jnp.dot(wt, s_tiled, preferred_element_type=f32)
        z = hu + kv + sw
        upd = h + g4 * (jnp.tanh(z) + 1.0)
        sq = jnp.dot(ones, upd * upd, preferred_element_type=f32)
        upd = upd * lax.rsqrt(jnp.maximum(sq, 1e-12))
        return jnp.where(m_t > 0.5, upd, h)

    o_ref[0] = lax.fori_loop(0, S, step, kt)


@jax.jit
def kernel(encoded_sents, mask, keys, U, V, W):
    f32 = jnp.float32
    s_t = (jnp.transpose(encoded_sents, (1, 2, 0))
           .reshape(S, D, NBLK, BB).transpose(0, 2, 1, 3))
    m_t = jnp.broadcast_to(
        jnp.swapaxes(mask, 0, 1).astype(f32).reshape(S, NBLK, 1, 1, BB),
        (S, NBLK, 1, K, BB)).reshape(S, NBLK, 1, N)
    k_t = (jnp.transpose(keys, (2, 1, 0))
           .reshape(D, K, NBLK, BB).transpose(2, 0, 1, 3)
           .reshape(NBLK, D, N))
    grid = (NBLK,)
    out_t = pl.pallas_call(
        _body,
        grid=grid,
        in_specs=[
            pl.BlockSpec((S, 1, D, BB), lambda i: (0, i, 0, 0)),
            pl.BlockSpec((S, 1, 1, N), lambda i: (0, i, 0, 0)),
            pl.BlockSpec((1, D, N), lambda i: (i, 0, 0)),
            pl.BlockSpec((D, D), lambda i: (0, 0)),
            pl.BlockSpec((D, D), lambda i: (0, 0)),
            pl.BlockSpec((D, D), lambda i: (0, 0)),
        ],
        out_specs=pl.BlockSpec((1, D, N), lambda i: (i, 0, 0)),
        out_shape=jax.ShapeDtypeStruct((NBLK, D, N), f32),
    )(s_t, m_t, k_t, 0.5 * U.T, 0.5 * V.T, 0.5 * W.T)
    return (out_t.reshape(NBLK, D, K, BB).transpose(0, 3, 2, 1)
            .reshape(B, K, D))


# speedup vs baseline: 1.6005x; 1.5669x over previous
import jax
import jax.numpy as jnp
from jax import lax
from jax.experimental import pallas as pl
from jax.experimental.pallas import tpu as pltpu

B, S, K, D = 4096, 20, 64, 32
BB = 128
NBLK = B // BB
N = K * BB
CC = 256
KC = CC // BB
NC = N // CC


def _body(s_ref, m_ref, k_ref, ut_ref, vt_ref, wt_ref, o_ref, h_ref, kv_ref):
    f32 = jnp.float32
    ut = ut_ref[...]
    vt = vt_ref[...]
    wt = wt_ref[...]
    kt_full = k_ref[0]
    kv_ref[...] = jnp.dot(vt, kt_full, preferred_element_type=f32)
    h_ref[...] = kt_full

    def step(t, carry):
        s_t = s_ref[pl.ds(t, 1)].reshape(D, BB)
        m_t = m_ref[pl.ds(t, 1)].reshape(1, BB)
        sw_t = jnp.dot(wt, s_t, preferred_element_type=f32)
        s_rep = jnp.concatenate([s_t] * KC, axis=1)
        sw_rep = jnp.concatenate([sw_t] * KC, axis=1)
        m_rep = jnp.concatenate([m_t] * KC, axis=1)
        for c in range(NC):
            sl = pl.ds(c * CC, CC)
            h = h_ref[:, sl]
            kt = k_ref[0, :, sl]
            g_half = 0.5 * jnp.sum(s_rep * (h + kt), axis=0, keepdims=True)
            g4 = 0.25 * jnp.tanh(g_half) + 0.25
            hu = jnp.dot(ut, h, preferred_element_type=f32)
            z = hu + kv_ref[:, sl] + sw_rep
            upd = h + g4 * (jnp.tanh(z) + 1.0)
            sq = jnp.sum(upd * upd, axis=0, keepdims=True)
            upd = upd * lax.rsqrt(jnp.maximum(sq, 1e-12))
            h_ref[:, sl] = jnp.where(m_rep > 0.5, upd, h)
        return carry

    lax.fori_loop(0, S, step, 0)
    o_ref[0] = h_ref[...]


@jax.jit
def kernel(encoded_sents, mask, keys, U, V, W):
    f32 = jnp.float32
    s_t = (jnp.transpose(encoded_sents, (1, 2, 0))
           .reshape(S, D, NBLK, BB).transpose(0, 2, 1, 3))
    m_t = jnp.swapaxes(mask, 0, 1).astype(f32).reshape(S, NBLK, 1, BB)
    k_t = (jnp.transpose(keys, (2, 1, 0))
           .reshape(D, K, NBLK, BB).transpose(2, 0, 1, 3)
           .reshape(NBLK, D, N))
    grid = (NBLK,)
    out_t = pl.pallas_call(
        _body,
        grid=grid,
        in_specs=[
            pl.BlockSpec((S, 1, D, BB), lambda i: (0, i, 0, 0)),
            pl.BlockSpec((S, 1, 1, BB), lambda i: (0, i, 0, 0)),
            pl.BlockSpec((1, D, N), lambda i: (i, 0, 0)),
            pl.BlockSpec((D, D), lambda i: (0, 0)),
            pl.BlockSpec((D, D), lambda i: (0, 0)),
            pl.BlockSpec((D, D), lambda i: (0, 0)),
        ],
        out_specs=pl.BlockSpec((1, D, N), lambda i: (i, 0, 0)),
        out_shape=jax.ShapeDtypeStruct((NBLK, D, N), f32),
        scratch_shapes=[
            pltpu.VMEM((D, N), f32),
            pltpu.VMEM((D, N), f32),
        ],
    )(s_t, m_t, k_t, 0.5 * U.T, 0.5 * V.T, 0.5 * W.T)
    return (out_t.reshape(NBLK, D, K, BB).transpose(0, 3, 2, 1)
            .reshape(B, K, D))


# speedup vs baseline: 1.9786x; 1.2362x over previous
import jax
import jax.numpy as jnp
from jax import lax
from jax.experimental import pallas as pl
from jax.experimental.pallas import tpu as pltpu

B, S, K, D = 4096, 20, 64, 32
BB = 128
NBLK = B // BB


def _body(s_ref, m_ref, k_ref, ut_ref, vt_ref, wt_ref, o_ref,
          h_ref, kt_ref, kv_ref, st_ref, mt_ref):
    f32 = jnp.float32
    ut = ut_ref[...]
    vt = vt_ref[...]
    wt = wt_ref[...]

    kt_all = k_ref[...].T
    kt_ref[...] = kt_all
    h_ref[...] = kt_all
    for k in range(K):
        r = pl.ds(k * D, D)
        kv_ref[r, :] = jnp.dot(vt, kt_all[k * D:(k + 1) * D, :],
                               preferred_element_type=f32)
    st_all = s_ref[...].T
    mt_all = m_ref[...].T
    for t in range(S):
        st_ref[t] = st_all[t * D:(t + 1) * D, :]
        mt_ref[t] = mt_all[t:t + 1, :]

    def step(t, carry):
        s_t = st_ref[pl.ds(t, 1)].reshape(D, BB)
        m_t = mt_ref[pl.ds(t, 1)].reshape(1, BB) > 0.5
        sw_t = jnp.dot(wt, s_t, preferred_element_type=f32)
        for k in range(K):
            r = pl.ds(k * D, D)
            h = h_ref[r, :]
            g_half = 0.5 * jnp.sum(s_t * (h + kt_ref[r, :]),
                                   axis=0, keepdims=True)
            g4 = 0.25 * jnp.tanh(g_half) + 0.25
            hu = jnp.dot(ut, h, preferred_element_type=f32)
            z = hu + kv_ref[r, :] + sw_t
            upd = h + g4 * (jnp.tanh(z) + 1.0)
            sq = jnp.sum(upd * upd, axis=0, keepdims=True)
            upd = upd * lax.rsqrt(jnp.maximum(sq, 1e-12))
            h_ref[r, :] = jnp.where(m_t, upd, h)
        return carry

    lax.fori_loop(0, S, step, 0)
    o_ref[...] = h_ref[...].T


@jax.jit
def kernel(encoded_sents, mask, keys, U, V, W):
    f32 = jnp.float32
    grid = (NBLK,)
    out = pl.pallas_call(
        _body,
        grid=grid,
        in_specs=[
            pl.BlockSpec((BB, S * D), lambda i: (i, 0)),
            pl.BlockSpec((BB, S), lambda i: (i, 0)),
            pl.BlockSpec((BB, K * D), lambda i: (i, 0)),
            pl.BlockSpec((D, D), lambda i: (0, 0)),
            pl.BlockSpec((D, D), lambda i: (0, 0)),
            pl.BlockSpec((D, D), lambda i: (0, 0)),
        ],
        out_specs=pl.BlockSpec((BB, K * D), lambda i: (i, 0)),
        out_shape=jax.ShapeDtypeStruct((B, K * D), f32),
        scratch_shapes=[
            pltpu.VMEM((K * D, BB), f32),
            pltpu.VMEM((K * D, BB), f32),
            pltpu.VMEM((K * D, BB), f32),
            pltpu.VMEM((S, D, BB), f32),
            pltpu.VMEM((S, 1, BB), f32),
        ],
    )(encoded_sents.reshape(B, S * D), mask.astype(f32),
      keys.reshape(B, K * D), 0.5 * U.T, 0.5 * V.T, 0.5 * W.T)
    return out.reshape(B, K, D)


# speedup vs baseline: 1.9941x; 1.0078x over previous
import jax
import jax.numpy as jnp
from jax import lax
from jax.experimental import pallas as pl
from jax.experimental.pallas import tpu as pltpu

B, S, K, D = 4096, 20, 64, 32
BB = 128
NBLK = B // BB


def _body(s_ref, m_ref, k_ref, uvt_ref, wt_ref, o_ref,
          h_ref, kt_ref, st_ref, mt_ref):
    f32 = jnp.float32
    uvt = uvt_ref[...]
    wt = wt_ref[...]

    kt_all = k_ref[...].T
    kt_ref[...] = kt_all
    h_ref[...] = kt_all
    st_all = s_ref[...].T
    mt_all = m_ref[...].T
    for t in range(S):
        st_ref[t] = st_all[t * D:(t + 1) * D, :]
        mt_ref[t] = mt_all[t:t + 1, :]

    def step(t, carry):
        s_t = st_ref[pl.ds(t, 1)].reshape(D, BB)
        m_t = mt_ref[pl.ds(t, 1)].reshape(1, BB) > 0.5
        sw_t = jnp.dot(wt, s_t, preferred_element_type=f32)
        for k in range(K):
            r = pl.ds(k * D, D)
            h = h_ref[r, :]
            kt = kt_ref[r, :]
            g_half = 0.5 * jnp.sum(s_t * (h + kt), axis=0, keepdims=True)
            g4 = 0.25 * jnp.tanh(g_half) + 0.25
            huv = jnp.dot(uvt, jnp.concatenate([h, kt], axis=0),
                          preferred_element_type=f32)
            z = huv + sw_t
            upd = h + g4 * (jnp.tanh(z) + 1.0)
            sq = jnp.sum(upd * upd, axis=0, keepdims=True)
            upd = upd * lax.rsqrt(jnp.maximum(sq, 1e-12))
            h_ref[r, :] = jnp.where(m_t, upd, h)
        return carry

    lax.fori_loop(0, S, step, 0)
    o_ref[...] = h_ref[...].T


@jax.jit
def kernel(encoded_sents, mask, keys, U, V, W):
    f32 = jnp.float32
    grid = (NBLK,)
    out = pl.pallas_call(
        _body,
        grid=grid,
        in_specs=[
            pl.BlockSpec((BB, S * D), lambda i: (i, 0)),
            pl.BlockSpec((BB, S), lambda i: (i, 0)),
            pl.BlockSpec((BB, K * D), lambda i: (i, 0)),
            pl.BlockSpec((D, 2 * D), lambda i: (0, 0)),
            pl.BlockSpec((D, D), lambda i: (0, 0)),
        ],
        out_specs=pl.BlockSpec((BB, K * D), lambda i: (i, 0)),
        out_shape=jax.ShapeDtypeStruct((B, K * D), f32),
        scratch_shapes=[
            pltpu.VMEM((K * D, BB), f32),
            pltpu.VMEM((K * D, BB), f32),
            pltpu.VMEM((S, D, BB), f32),
            pltpu.VMEM((S, 1, BB), f32),
        ],
    )(encoded_sents.reshape(B, S * D), mask.astype(f32),
      keys.reshape(B, K * D),
      0.5 * jnp.concatenate([U.T, V.T], axis=1), 0.5 * W.T)
    return out.reshape(B, K, D)


# speedup vs baseline: 2.0365x; 1.0212x over previous
import jax
import jax.numpy as jnp
from jax import lax
from jax.experimental import pallas as pl
from jax.experimental.pallas import tpu as pltpu

B, S, K, D = 4096, 20, 64, 32
BB = 256
NBLK = B // BB


def _body(s_ref, m_ref, k_ref, uvt_ref, wt_ref, o_ref,
          h_ref, kt_ref, st_ref, mt_ref):
    f32 = jnp.float32
    uvt = uvt_ref[...]
    wt = wt_ref[...]

    kt_all = k_ref[...].T
    kt_ref[...] = kt_all
    h_ref[...] = kt_all
    st_all = s_ref[...].T
    mt_all = m_ref[...].T
    for t in range(S):
        st_ref[t] = st_all[t * D:(t + 1) * D, :]
        mt_ref[t] = mt_all[t:t + 1, :]

    def step(t, carry):
        s_t = st_ref[pl.ds(t, 1)].reshape(D, BB)
        m_t = mt_ref[pl.ds(t, 1)].reshape(1, BB) > 0.5
        sw_t = jnp.dot(wt, s_t, preferred_element_type=f32)
        for k in range(K):
            r = pl.ds(k * D, D)
            h = h_ref[r, :]
            kt = kt_ref[r, :]
            g_half = 0.5 * jnp.sum(s_t * (h + kt), axis=0, keepdims=True)
            g4 = 0.25 * jnp.tanh(g_half) + 0.25
            huv = jnp.dot(uvt, jnp.concatenate([h, kt], axis=0),
                          preferred_element_type=f32)
            z = huv + sw_t
            upd = h + g4 * (jnp.tanh(z) + 1.0)
            sq = jnp.sum(upd * upd, axis=0, keepdims=True)
            upd = upd * lax.rsqrt(jnp.maximum(sq, 1e-12))
            h_ref[r, :] = jnp.where(m_t, upd, h)
        return carry

    lax.fori_loop(0, S, step, 0)
    o_ref[...] = h_ref[...].T


@jax.jit
def kernel(encoded_sents, mask, keys, U, V, W):
    f32 = jnp.float32
    grid = (NBLK,)
    out = pl.pallas_call(
        _body,
        grid=grid,
        in_specs=[
            pl.BlockSpec((BB, S * D), lambda i: (i, 0)),
            pl.BlockSpec((BB, S), lambda i: (i, 0)),
            pl.BlockSpec((BB, K * D), lambda i: (i, 0)),
            pl.BlockSpec((D, 2 * D), lambda i: (0, 0)),
            pl.BlockSpec((D, D), lambda i: (0, 0)),
        ],
        out_specs=pl.BlockSpec((BB, K * D), lambda i: (i, 0)),
        out_shape=jax.ShapeDtypeStruct((B, K * D), f32),
        scratch_shapes=[
            pltpu.VMEM((K * D, BB), f32),
            pltpu.VMEM((K * D, BB), f32),
            pltpu.VMEM((S, D, BB), f32),
            pltpu.VMEM((S, 1, BB), f32),
        ],
    )(encoded_sents.reshape(B, S * D), mask.astype(f32),
      keys.reshape(B, K * D),
      0.5 * jnp.concatenate([U.T, V.T], axis=1), 0.5 * W.T)
    return out.reshape(B, K, D)
